# baseline (speedup 1.0000x reference)
import functools

import jax
import jax.numpy as jnp
import numpy as np
from jax import lax
from jax.experimental import pallas as pl
from jax.experimental.pallas import tpu as pltpu
from jax.experimental.pallas import tpu_sc as plsc

_INTMIN = np.int32(-(2**31))

_NC = 2
_NS = 16
_L = 16

_CT = 1664


_PHASE_SHIFTS = (22, 11, 0)
_PHASE_BITS = (10, 11, 11)
_BINS = 2048
_CHUNK = 1024
_UNROLL = 8


def _key_from_f32(v):
    m = plsc.bitcast(v, jnp.int32)
    return m ^ ((m >> 31) | _INTMIN)


def _f32_from_key(k):
    m = jnp.where(k < 0, k ^ _INTMIN, ~k)
    return plsc.bitcast(m, jnp.float32)


def _digit16(key, shift, bits):
    mask16 = ((1 << bits) - 1) << 4
    if shift >= 4:
        return lax.shift_right_logical(key, shift - 4) & mask16
    return lax.shift_left(key, 4 - shift) & mask16


def _sc_median_body(x_hbm, out_hbm, hist, buf0, buf1, outbuf, sem0, sem1):
    nb, n, c = x_hbm.shape
    csc = c - _CT
    rank0 = (n - 1) // 2
    wid = lax.axis_index("s") * _NC + lax.axis_index("c")
    ngroups = nb * csc // _L
    gpw = ngroups // (_NC * _NS)
    cgroups = csc // _L
    crecip = (1 << 20) // cgroups + 1
    nchunks = n // _CHUNK
    lane = lax.iota(jnp.int32, _L)
    ones = jnp.ones((_L,), jnp.int32)
    zeros = jnp.zeros((_L,), jnp.int32)
    bufs = (buf0, buf1)
    sems = (sem0, sem1)

    def group_body(g, carry):
        gid = wid * gpw + g
        bidx = lax.shift_right_logical(gid * crecip, 20)
        rem = gid - bidx * cgroups
        c0 = pl.multiple_of(_CT + lax.shift_left(rem, 4), _L)
        o0 = pl.multiple_of(lax.shift_left(rem, 4), _L)

        psel = jnp.zeros((_L,), jnp.int32)
        r = jnp.full((_L,), rank0, jnp.int32)

        for phase in range(3):
            shift = _PHASE_SHIFTS[phase]
            bits = _PHASE_BITS[phase]
            nbins = 1 << bits

            def zero_body(z, _):
                base = lax.shift_left(z, 4 + 3)
                for u in range(_UNROLL):
                    hist[pl.ds(base + u * _L, _L)] = zeros
                return 0

            lax.fori_loop(0, nbins // _UNROLL, zero_body, 0)

            def row_body(rr, _, buf=None):
                base = rr * _UNROLL
                for u in range(_UNROLL):
                    key = _key_from_f32(buf[base + u])
                    idx = _digit16(key, shift, bits) | lane
                    if phase == 0:
                        plsc.addupdate_scatter(hist, [idx], ones)
                    else:
                        pref = lax.shift_right_logical(key, shift + bits)
                        plsc.addupdate_scatter(hist, [idx], ones,
                                               mask=pref == psel)
                return 0

            cp = pltpu.async_copy(
                x_hbm.at[bidx, pl.ds(0, _CHUNK), pl.ds(c0, _L)], buf0, sem0)
            for k in range(nchunks):
                if k + 1 < nchunks:
                    nxt = pltpu.async_copy(
                        x_hbm.at[bidx, pl.ds((k + 1) * _CHUNK, _CHUNK),
                                 pl.ds(c0, _L)],
                        bufs[(k + 1) % 2], sems[(k + 1) % 2])
                cp.wait()
                lax.fori_loop(
                    0, _CHUNK // _UNROLL,
                    functools.partial(row_body, buf=bufs[k % 2]), 0)
                if k + 1 < nchunks:
                    cp = nxt

            def scan_body(d, sc):
                cum, dsel, rnew = sc
                base = lax.shift_left(d, 4 + 2)
                for u in range(4):
                    h = hist[pl.ds(base + u * _L, _L)]
                    newcum = cum + h
                    cond = (cum <= r) & (newcum > r)
                    dsel = jnp.where(cond, d * 4 + u, dsel)
                    rnew = jnp.where(cond, r - cum, rnew)
                    cum = newcum
                return cum, dsel, rnew

            _, dsel, rnew = lax.fori_loop(
                0, nbins // 4, scan_body, (zeros, zeros, zeros))
            psel = lax.shift_left(psel, bits) | dsel
            r = rnew

        outbuf[...] = _f32_from_key(psel)
        pltpu.sync_copy(outbuf, out_hbm.at[bidx, pl.ds(o0, _L)])
        return 0

    lax.fori_loop(0, gpw, group_body, 0)


def _sc_median(x):
    nb, n, c = x.shape
    mesh = plsc.VectorSubcoreMesh(
        core_axis_name="c", subcore_axis_name="s",
        num_cores=_NC, num_subcores=_NS)
    f = functools.partial(
        pl.kernel,
        out_type=jax.ShapeDtypeStruct((nb, c - _CT), jnp.float32),
        mesh=mesh,
        scratch_types=[
            pltpu.VMEM((_BINS * _L,), jnp.int32),
            pltpu.VMEM((_CHUNK, _L), jnp.float32),
            pltpu.VMEM((_CHUNK, _L), jnp.float32),
            pltpu.VMEM((_L,), jnp.float32),
            pltpu.SemaphoreType.DMA,
            pltpu.SemaphoreType.DMA,
        ],
        compiler_params=pltpu.CompilerParams(
            use_tc_tiling_on_sc=False, needs_layout_passes=False),
    )(_sc_median_body)
    return f(x)


_CB = 128


def _tc_median_body(x_ref, o_ref, ks_ref, *, rank):
    i = pl.program_id(1)
    m = lax.bitcast_convert_type(x_ref[0], jnp.int32)
    ks_ref[...] = jnp.where(m < 0, ~m ^ _INTMIN, m)
    cb = x_ref.shape[2]

    def step(_, carry):
        p, bitv = carry
        trial = p | bitv
        cnt = jnp.sum(
            (ks_ref[...] < (trial ^ _INTMIN)).astype(jnp.int32),
            axis=0,
            keepdims=True,
        )
        p = jnp.where(cnt <= rank, trial, p)
        return p, lax.shift_right_logical(bitv, 1)

    p0 = jnp.zeros((1, cb), jnp.int32)
    p, _ = lax.fori_loop(0, 32, step, (p0, _INTMIN))
    m_out = jnp.where(p < 0, p ^ _INTMIN, ~p)
    o_ref[pl.ds(i, 1), :] = lax.bitcast_convert_type(m_out, jnp.float32)


def _tc_median(x):
    nb, n, c = x.shape
    rank = (n - 1) // 2
    grid = (_CT // _CB, nb)
    return pl.pallas_call(
        functools.partial(_tc_median_body, rank=rank),
        grid=grid,
        in_specs=[
            pl.BlockSpec((1, n, _CB), lambda j, i: (i, 0, j)),
        ],
        out_specs=pl.BlockSpec((nb, _CB), lambda j, i: (0, j)),
        out_shape=jax.ShapeDtypeStruct((nb, _CT), jnp.float32),
        scratch_shapes=[pltpu.VMEM((n, _CB), jnp.int32)],
    )(x)


def kernel(x):
    out_sc = _sc_median(x)
    out_tc = _tc_median(x)
    return jnp.concatenate([out_tc, out_sc], axis=1)


# speedup vs baseline: 1.8971x; 1.8971x over previous
import functools

import jax
import jax.numpy as jnp
import numpy as np
from jax import lax
from jax.experimental import pallas as pl
from jax.experimental.pallas import tpu as pltpu
from jax.experimental.pallas import tpu_sc as plsc

_INTMIN = np.int32(-(2**31))

_NC = 2
_NS = 16
_L = 16

_CT = 1792


_PHASE_SHIFTS = (22, 11, 0)
_PHASE_BITS = (10, 11, 11)
_BINS = 2048
_CHUNK = 1024
_UNROLL = 8


def _key_from_f32(v):
    m = plsc.bitcast(v, jnp.int32)
    return m ^ ((m >> 31) | _INTMIN)


def _f32_from_key(k):
    m = jnp.where(k < 0, k ^ _INTMIN, ~k)
    return plsc.bitcast(m, jnp.float32)


def _digit16(key, shift, bits):
    mask16 = ((1 << bits) - 1) << 4
    if shift >= 4:
        return lax.shift_right_logical(key, shift - 4) & mask16
    return lax.shift_left(key, 4 - shift) & mask16


def _sc_median_body(x_hbm, out_hbm, hist, buf0, buf1, outbuf, sem0, sem1):
    nb, n, csc = x_hbm.shape
    rank0 = (n - 1) // 2
    wid = lax.axis_index("s") * _NC + lax.axis_index("c")
    ngroups = nb * csc // _L
    gpw = ngroups // (_NC * _NS)
    cgroups = csc // _L
    crecip = (1 << 20) // cgroups + 1
    nchunks = n // _CHUNK
    lane = lax.iota(jnp.int32, _L)
    ones = jnp.ones((_L,), jnp.int32)
    zeros = jnp.zeros((_L,), jnp.int32)
    bufs = (buf0, buf1)
    sems = (sem0, sem1)

    def group_body(g, carry):
        gid = wid * gpw + g
        bidx = lax.shift_right_logical(gid * crecip, 20)
        rem = gid - bidx * cgroups
        c0 = pl.multiple_of(lax.shift_left(rem, 4), _L)
        o0 = c0

        psel = jnp.zeros((_L,), jnp.int32)
        r = jnp.full((_L,), rank0, jnp.int32)

        for phase in range(3):
            shift = _PHASE_SHIFTS[phase]
            bits = _PHASE_BITS[phase]
            nbins = 1 << bits

            def zero_body(z, _):
                base = lax.shift_left(z, 4 + 3)
                for u in range(_UNROLL):
                    hist[pl.ds(base + u * _L, _L)] = zeros
                return 0

            lax.fori_loop(0, nbins // _UNROLL, zero_body, 0)

            def row_body(rr, _, buf=None):
                base = rr * _UNROLL
                for u in range(_UNROLL):
                    key = _key_from_f32(buf[base + u])
                    idx = _digit16(key, shift, bits) | lane
                    if phase == 0:
                        plsc.addupdate_scatter(hist, [idx], ones)
                    else:
                        pref = lax.shift_right_logical(key, shift + bits)
                        plsc.addupdate_scatter(hist, [idx], ones,
                                               mask=pref == psel)
                return 0

            cp = pltpu.async_copy(
                x_hbm.at[bidx, pl.ds(0, _CHUNK), pl.ds(c0, _L)], buf0, sem0)
            for k in range(nchunks):
                if k + 1 < nchunks:
                    nxt = pltpu.async_copy(
                        x_hbm.at[bidx, pl.ds((k + 1) * _CHUNK, _CHUNK),
                                 pl.ds(c0, _L)],
                        bufs[(k + 1) % 2], sems[(k + 1) % 2])
                cp.wait()
                lax.fori_loop(
                    0, _CHUNK // _UNROLL,
                    functools.partial(row_body, buf=bufs[k % 2]), 0)
                if k + 1 < nchunks:
                    cp = nxt

            def scan_body(d, sc):
                cum, dsel, rnew = sc
                base = lax.shift_left(d, 4 + 2)
                for u in range(4):
                    h = hist[pl.ds(base + u * _L, _L)]
                    newcum = cum + h
                    cond = (cum <= r) & (newcum > r)
                    dsel = jnp.where(cond, d * 4 + u, dsel)
                    rnew = jnp.where(cond, r - cum, rnew)
                    cum = newcum
                return cum, dsel, rnew

            _, dsel, rnew = lax.fori_loop(
                0, nbins // 4, scan_body, (zeros, zeros, zeros))
            psel = lax.shift_left(psel, bits) | dsel
            r = rnew

        outbuf[...] = _f32_from_key(psel)
        pltpu.sync_copy(outbuf, out_hbm.at[bidx, pl.ds(o0, _L)])
        return 0

    lax.fori_loop(0, gpw, group_body, 0)


def _sc_median(x_sc):
    nb, n, csc = x_sc.shape
    mesh = plsc.VectorSubcoreMesh(
        core_axis_name="c", subcore_axis_name="s",
        num_cores=_NC, num_subcores=_NS)
    f = functools.partial(
        pl.kernel,
        out_type=jax.ShapeDtypeStruct((nb, csc), jnp.float32),
        mesh=mesh,
        scratch_types=[
            pltpu.VMEM((_BINS * _L,), jnp.int32),
            pltpu.VMEM((_CHUNK, _L), jnp.float32),
            pltpu.VMEM((_CHUNK, _L), jnp.float32),
            pltpu.VMEM((_L,), jnp.float32),
            pltpu.SemaphoreType.DMA,
            pltpu.SemaphoreType.DMA,
        ],
        compiler_params=pltpu.CompilerParams(
            use_tc_tiling_on_sc=False, needs_layout_passes=False),
    )(_sc_median_body)
    return f(x_sc)


_CB = 256


def _tc_median_body(x_ref, o_ref, ks_ref, *, rank):
    i = pl.program_id(1)
    m = lax.bitcast_convert_type(x_ref[0], jnp.int32)
    ks_ref[...] = jnp.where(m < 0, ~m ^ _INTMIN, m)
    cb = x_ref.shape[2]

    def step(_, carry):
        p, bitv = carry
        trial = p | bitv
        cnt = jnp.sum(
            (ks_ref[...] < (trial ^ _INTMIN)).astype(jnp.int32),
            axis=0,
            keepdims=True,
        )
        p = jnp.where(cnt <= rank, trial, p)
        return p, lax.shift_right_logical(bitv, 1)

    p0 = jnp.zeros((1, cb), jnp.int32)
    p, _ = lax.fori_loop(0, 32, step, (p0, _INTMIN))
    m_out = jnp.where(p < 0, p ^ _INTMIN, ~p)
    o_ref[pl.ds(i, 1), :] = lax.bitcast_convert_type(m_out, jnp.float32)


def _tc_median(x):
    nb, n, c = x.shape
    rank = (n - 1) // 2
    grid = (_CT // _CB, nb)
    return pl.pallas_call(
        functools.partial(_tc_median_body, rank=rank),
        grid=grid,
        in_specs=[
            pl.BlockSpec((1, n, _CB), lambda j, i: (i, 0, j)),
        ],
        out_specs=pl.BlockSpec((nb, _CB), lambda j, i: (0, j)),
        out_shape=jax.ShapeDtypeStruct((nb, _CT), jnp.float32),
        scratch_shapes=[pltpu.VMEM((n, _CB), jnp.int32)],
    )(x)


def kernel(x):
    out_sc = _sc_median(x[:, :, _CT:])
    out_tc = _tc_median(x)
    return jnp.concatenate([out_tc, out_sc], axis=1)
